# baseline (speedup 1.0000x reference)
import functools
import jax
import jax.numpy as jnp
from jax.experimental import pallas as pl
from jax.experimental.pallas import tpu as pltpu
from jax.experimental.pallas import tpu_sc as plsc

D_MODEL = 1024
D_FF = 4096
E = 8
T = 2048
A = 2 * T
TM = 128
PAD = A + E * TM
NT = PAD // TM
TF = 1024
NF = D_FF // TF
AR = 32
NCHUNK = 512

NC = 2
NS = 16
NW = NC * NS

_INV_SQRT2 = 0.7071067811865476


def _gelu_exact(v):
    return 0.5 * v * (1.0 + jax.lax.erf(v * _INV_SQRT2))


def _router_body(x_ref, wr_ref, br_ref, slots_ref, g0_ref, g1_ref, eot_ref):
    logits = jnp.dot(x_ref[...], wr_ref[...], preferred_element_type=jnp.float32)
    logits = logits + br_ref[0][None, :]
    lane = jax.lax.broadcasted_iota(jnp.int32, logits.shape, 1)
    s0 = jnp.max(logits, axis=1, keepdims=True)
    i0 = jnp.min(jnp.where(logits == s0, lane, E), axis=1, keepdims=True)
    masked = jnp.where(lane == i0, -jnp.inf, logits)
    s1 = jnp.max(masked, axis=1, keepdims=True)
    i1 = jnp.min(jnp.where(masked == s1, lane, E), axis=1, keepdims=True)
    ee = jnp.exp(s1 - s0)
    g0 = 1.0 / (1.0 + ee)
    g1 = ee / (1.0 + ee)

    ef = jnp.concatenate(
        [jnp.reshape(i0, (AR // 2, 128)), jnp.reshape(i1, (AR // 2, 128))], axis=0
    ).astype(jnp.float32)
    g0_ref[...] = g0
    g1_ref[...] = g1

    ci = jax.lax.broadcasted_iota(jnp.int32, (128, 128), 0)
    cj = jax.lax.broadcasted_iota(jnp.int32, (128, 128), 1)
    lt128 = (ci < cj).astype(jnp.float32)
    ri = jax.lax.broadcasted_iota(jnp.int32, (AR, AR), 0)
    rj = jax.lax.broadcasted_iota(jnp.int32, (AR, AR), 1)
    lt_rows = (rj < ri).astype(jnp.float32)
    ones_col = jnp.ones((128, 1), jnp.float32)

    slots = jnp.zeros((AR, 128), jnp.float32)
    offset = 0.0
    ends = []
    for e in range(E):
        m = (ef == float(e)).astype(jnp.float32)
        rank_in_row = jnp.dot(m, lt128, preferred_element_type=jnp.float32)
        row_sums = jnp.dot(m, ones_col, preferred_element_type=jnp.float32)
        row_prefix = jnp.dot(lt_rows, row_sums, preferred_element_type=jnp.float32)
        rank = rank_in_row + row_prefix
        count = jnp.sum(row_sums)
        slots = slots + m * (offset + rank)
        padded = jnp.ceil(count / TM) * TM
        offset = offset + padded
        ends.append(offset)

    slots_ref[...] = slots.astype(jnp.int32)

    jt = jax.lax.broadcasted_iota(jnp.int32, (1, 128), 1).astype(jnp.float32) * float(TM)
    eot = jnp.zeros((1, 128), jnp.float32)
    for e in range(E):
        eot = eot + (jt >= ends[e]).astype(jnp.float32)
    eot_ref[...] = eot.astype(jnp.int32)


def _run_router(flat_x, Wr, br2):
    return pl.pallas_call(
        _router_body,
        out_shape=(
            jax.ShapeDtypeStruct((AR, 128), jnp.int32),
            jax.ShapeDtypeStruct((T, 1), jnp.float32),
            jax.ShapeDtypeStruct((T, 1), jnp.float32),
            jax.ShapeDtypeStruct((1, 128), jnp.int32),
        ),
        in_specs=[
            pl.BlockSpec((T, D_MODEL), lambda: (0, 0)),
            pl.BlockSpec((D_MODEL, E), lambda: (0, 0)),
            pl.BlockSpec((1, E), lambda: (0, 0)),
        ],
        out_specs=(
            pl.BlockSpec((AR, 128), lambda: (0, 0)),
            pl.BlockSpec((T, 1), lambda: (0, 0)),
            pl.BlockSpec((T, 1), lambda: (0, 0)),
            pl.BlockSpec((1, 128), lambda: (0, 0)),
        ),
    )(flat_x, Wr, br2)


def _invert_body(slots_ref, tok_ref):
    j = pl.program_id(0)
    s_col = slots_ref[...].astype(jnp.float32)
    sid = (jax.lax.broadcasted_iota(jnp.int32, (1, NCHUNK), 1)
           + j * NCHUNK).astype(jnp.float32)
    onehot = (s_col == sid).astype(jnp.float32)
    toks = (jax.lax.broadcasted_iota(jnp.int32, (1, A), 1) % T).astype(jnp.float32)
    tok = jnp.dot(toks, onehot, preferred_element_type=jnp.float32)
    tok_ref[...] = tok.astype(jnp.int32)


def _run_invert(slots_col):
    return pl.pallas_call(
        _invert_body,
        grid=(PAD // NCHUNK,),
        out_shape=jax.ShapeDtypeStruct((1, PAD), jnp.int32),
        in_specs=[pl.BlockSpec((A, 1), lambda j: (0, 0))],
        out_specs=pl.BlockSpec((1, NCHUNK), lambda j: (0, j)),
    )(slots_col)


def _make_sc_gather(n_rows, n_chunks=4):
    bpw = n_rows // NW
    chunk = bpw // n_chunks
    mesh = plsc.VectorSubcoreMesh(core_axis_name="c", subcore_axis_name="s")

    @functools.partial(
        pl.kernel,
        mesh=mesh,
        out_type=jax.ShapeDtypeStruct((n_rows, D_MODEL), jnp.float32),
        scratch_types=[
            pltpu.VMEM((bpw,), jnp.int32),
            pltpu.VMEM((chunk, D_MODEL), jnp.float32),
            pltpu.VMEM((chunk, D_MODEL), jnp.float32),
            pltpu.SemaphoreType.DMA,
            pltpu.SemaphoreType.DMA,
        ],
    )
    def gather(table_hbm, idx_hbm, out_hbm, idx_v, r0, r1, s0, s1):
        wid = jax.lax.axis_index("s") * NC + jax.lax.axis_index("c")
        base = wid * bpw
        pltpu.sync_copy(idx_hbm.at[pl.ds(base, bpw)], idx_v)
        bufs, sems = (r0, r1), (s0, s1)

        def fire(c):
            return pltpu.async_copy(
                table_hbm.at[idx_v.at[pl.ds(c * chunk, chunk)]],
                bufs[c % 2], sems[c % 2])

        descs = {0: fire(0)}
        if n_chunks > 1:
            descs[1] = fire(1)
        for c in range(n_chunks):
            descs[c].wait()
            pltpu.sync_copy(bufs[c % 2], out_hbm.at[pl.ds(base + c * chunk, chunk)])
            if c + 2 < n_chunks:
                descs[c + 2] = fire(c + 2)

    return gather


def _sc_gather_rows(table, idx, n_rows):
    return _make_sc_gather(n_rows)(table, idx)


def _make_sc_dispatch():
    bpw = A // NW
    n_chunks = 4
    chunk = bpw // n_chunks
    mesh = plsc.VectorSubcoreMesh(core_axis_name="c", subcore_axis_name="s")

    @functools.partial(
        pl.kernel,
        mesh=mesh,
        out_type=jax.ShapeDtypeStruct((PAD, D_MODEL), jnp.float32),
        scratch_types=[
            pltpu.VMEM((chunk,), jnp.int32),
            pltpu.VMEM((chunk,), jnp.int32),
            pltpu.VMEM((chunk, D_MODEL), jnp.float32),
            pltpu.VMEM((chunk, D_MODEL), jnp.float32),
            pltpu.SemaphoreType.DMA,
            pltpu.SemaphoreType.DMA,
        ],
    )
    def dispatch(x_hbm, slots_hbm, out_hbm, i0, i1, r0, r1, s0, s1):
        wid = jax.lax.axis_index("s") * NC + jax.lax.axis_index("c")
        a_base = wid * bpw
        tok_base = jax.lax.rem(a_base, T)
        ibufs, rbufs, sems = (i0, i1), (r0, r1), (s0, s1)

        def load_and_fire(c):
            ib, rb, sem = ibufs[c % 2], rbufs[c % 2], sems[c % 2]
            pltpu.sync_copy(slots_hbm.at[pl.ds(a_base + c * chunk, chunk)], ib)
            pltpu.sync_copy(x_hbm.at[pl.ds(tok_base + c * chunk, chunk)], rb)
            return pltpu.async_copy(rb, out_hbm.at[ib], sem)

        descs = {0: load_and_fire(0), 1: load_and_fire(1)}
        for c in range(n_chunks):
            descs[c].wait()
            if c + 2 < n_chunks:
                descs[c + 2] = load_and_fire(c + 2)

    return dispatch


def _ffn_body(eot_ref, x_ref, w1_ref, b1_ref, w2_ref, b2_ref, out_ref):
    f = pl.program_id(0)
    mt = pl.program_id(1)

    @pl.when(eot_ref[mt] < E)
    def _():
        x = x_ref[...]
        h = jnp.dot(x, w1_ref[0], preferred_element_type=jnp.float32) + b1_ref[0]
        h = _gelu_exact(h)
        contrib = jnp.dot(h, w2_ref[0], preferred_element_type=jnp.float32)
        contrib = contrib + jnp.where(f == 0, 1.0, 0.0) * b2_ref[0]
        sl = pl.ds(mt * TM, TM)
        prev = jnp.where(f == 0, 0.0, out_ref[sl, :])
        out_ref[sl, :] = prev + contrib


def _run_ffn(x_pad, W1, b13, W2, b23, eot):
    def _e(eot_ref, mt):
        return jnp.minimum(eot_ref[mt], E - 1)

    grid_spec = pltpu.PrefetchScalarGridSpec(
        num_scalar_prefetch=1,
        grid=(NF, NT),
        in_specs=[
            pl.BlockSpec((TM, D_MODEL), lambda f, mt, eot: (mt, 0)),
            pl.BlockSpec((1, D_MODEL, TF), lambda f, mt, eot: (_e(eot, mt), 0, f)),
            pl.BlockSpec((1, 1, TF), lambda f, mt, eot: (_e(eot, mt), 0, f)),
            pl.BlockSpec((1, TF, D_MODEL), lambda f, mt, eot: (_e(eot, mt), f, 0)),
            pl.BlockSpec((1, 1, D_MODEL), lambda f, mt, eot: (_e(eot, mt), 0, 0)),
        ],
        out_specs=pl.BlockSpec((PAD, D_MODEL), lambda f, mt, eot: (0, 0)),
    )
    return pl.pallas_call(
        _ffn_body,
        grid_spec=grid_spec,
        out_shape=jax.ShapeDtypeStruct((PAD, D_MODEL), jnp.float32),
        compiler_params=pltpu.CompilerParams(
            dimension_semantics=("arbitrary", "arbitrary"),
        ),
    )(eot, x_pad, W1, b13, W2, b23)


def _combine_body(y0_ref, y1_ref, g0_ref, g1_ref, out_ref):
    out_ref[...] = g0_ref[...] * y0_ref[...] + g1_ref[...] * y1_ref[...]


def _run_combine(yg, g0, g1):
    nt = T // TM
    return pl.pallas_call(
        _combine_body,
        grid=(nt,),
        out_shape=jax.ShapeDtypeStruct((T, D_MODEL), jnp.float32),
        in_specs=[
            pl.BlockSpec((TM, D_MODEL), lambda mt: (mt, 0)),
            pl.BlockSpec((TM, D_MODEL), lambda mt: (mt + nt, 0)),
            pl.BlockSpec((TM, 1), lambda mt: (mt, 0)),
            pl.BlockSpec((TM, 1), lambda mt: (mt, 0)),
        ],
        out_specs=pl.BlockSpec((TM, D_MODEL), lambda mt: (mt, 0)),
    )(yg, yg, g0, g1)


def kernel(x, Wr, br, W1, b1, W2, b2):
    bsz, seq_len, d_model = x.shape
    flat_x = x.reshape(-1, d_model)

    slots32, g0, g1, eot128 = _run_router(flat_x, Wr, br.reshape(1, E))

    x_pad = _make_sc_dispatch()(flat_x, slots32.reshape(A))
    y_pad = _run_ffn(x_pad, W1, b1.reshape(E, 1, D_FF),
                     W2, b2.reshape(E, 1, D_MODEL),
                     eot128.reshape(128))

    yg = _sc_gather_rows(y_pad, slots32.reshape(A), A)
    out = _run_combine(yg, g0, g1)
    return out.reshape(bsz, seq_len, d_model)


# speedup vs baseline: 1.1929x; 1.1929x over previous
import functools
import jax
import jax.numpy as jnp
from jax.experimental import pallas as pl
from jax.experimental.pallas import tpu as pltpu
from jax.experimental.pallas import tpu_sc as plsc

D_MODEL = 1024
D_FF = 4096
E = 8
T = 2048
A = 2 * T
TM = 128
PAD = A + E * TM
NT = PAD // TM
TF = 2048
NF = D_FF // TF
AR = 32

NC = 2
NS = 16
NW = NC * NS

_INV_SQRT2 = 0.7071067811865476


def _gelu_exact(v):
    return 0.5 * v * (1.0 + jax.lax.erf(v * _INV_SQRT2))


def _router_body(x_ref, wr_ref, br_ref, slots_ref, g0_ref, g1_ref, eot_ref):
    logits = jnp.dot(x_ref[...], wr_ref[...], preferred_element_type=jnp.float32)
    logits = logits + br_ref[0][None, :]
    lane = jax.lax.broadcasted_iota(jnp.int32, logits.shape, 1)
    s0 = jnp.max(logits, axis=1, keepdims=True)
    i0 = jnp.min(jnp.where(logits == s0, lane, E), axis=1, keepdims=True)
    masked = jnp.where(lane == i0, -jnp.inf, logits)
    s1 = jnp.max(masked, axis=1, keepdims=True)
    i1 = jnp.min(jnp.where(masked == s1, lane, E), axis=1, keepdims=True)
    ee = jnp.exp(s1 - s0)
    g0 = 1.0 / (1.0 + ee)
    g1 = ee / (1.0 + ee)

    ef = jnp.concatenate(
        [jnp.reshape(i0, (AR // 2, 128)), jnp.reshape(i1, (AR // 2, 128))], axis=0
    ).astype(jnp.float32)
    g0_ref[...] = g0
    g1_ref[...] = g1

    ci = jax.lax.broadcasted_iota(jnp.int32, (128, 128), 0)
    cj = jax.lax.broadcasted_iota(jnp.int32, (128, 128), 1)
    lt128 = (ci < cj).astype(jnp.float32)
    ri = jax.lax.broadcasted_iota(jnp.int32, (AR, AR), 0)
    rj = jax.lax.broadcasted_iota(jnp.int32, (AR, AR), 1)
    lt_rows = (rj < ri).astype(jnp.float32)
    ones_col = jnp.ones((128, 1), jnp.float32)

    slots = jnp.zeros((AR, 128), jnp.float32)
    offset = 0.0
    ends = []
    for e in range(E):
        m = (ef == float(e)).astype(jnp.float32)
        rank_in_row = jnp.dot(m, lt128, preferred_element_type=jnp.float32)
        row_sums = jnp.dot(m, ones_col, preferred_element_type=jnp.float32)
        row_prefix = jnp.dot(lt_rows, row_sums, preferred_element_type=jnp.float32)
        rank = rank_in_row + row_prefix
        count = jnp.sum(row_sums)
        slots = slots + m * (offset + rank)
        padded = jnp.ceil(count / TM) * TM
        offset = offset + padded
        ends.append(offset)

    slots_ref[...] = slots.astype(jnp.int32)

    jt = jax.lax.broadcasted_iota(jnp.int32, (1, 128), 1).astype(jnp.float32) * float(TM)
    eot = jnp.zeros((1, 128), jnp.float32)
    for e in range(E):
        eot = eot + (jt >= ends[e]).astype(jnp.float32)
    eot_ref[...] = eot.astype(jnp.int32)


def _run_router(flat_x, Wr, br2):
    return pl.pallas_call(
        _router_body,
        out_shape=(
            jax.ShapeDtypeStruct((AR, 128), jnp.int32),
            jax.ShapeDtypeStruct((T, 1), jnp.float32),
            jax.ShapeDtypeStruct((T, 1), jnp.float32),
            jax.ShapeDtypeStruct((1, 128), jnp.int32),
        ),
        in_specs=[
            pl.BlockSpec((T, D_MODEL), lambda: (0, 0)),
            pl.BlockSpec((D_MODEL, E), lambda: (0, 0)),
            pl.BlockSpec((1, E), lambda: (0, 0)),
        ],
        out_specs=(
            pl.BlockSpec((AR, 128), lambda: (0, 0)),
            pl.BlockSpec((T, 1), lambda: (0, 0)),
            pl.BlockSpec((T, 1), lambda: (0, 0)),
            pl.BlockSpec((1, 128), lambda: (0, 0)),
        ),
    )(flat_x, Wr, br2)


def _make_sc_gather(n_rows, n_chunks=4):
    bpw = n_rows // NW
    chunk = bpw // n_chunks
    mesh = plsc.VectorSubcoreMesh(core_axis_name="c", subcore_axis_name="s")

    @functools.partial(
        pl.kernel,
        mesh=mesh,
        out_type=jax.ShapeDtypeStruct((n_rows, D_MODEL), jnp.float32),
        scratch_types=[
            pltpu.VMEM((bpw,), jnp.int32),
            pltpu.VMEM((chunk, D_MODEL), jnp.float32),
            pltpu.VMEM((chunk, D_MODEL), jnp.float32),
            pltpu.SemaphoreType.DMA,
            pltpu.SemaphoreType.DMA,
        ],
    )
    def gather(table_hbm, idx_hbm, out_hbm, idx_v, r0, r1, s0, s1):
        wid = jax.lax.axis_index("s") * NC + jax.lax.axis_index("c")
        base = wid * bpw
        pltpu.sync_copy(idx_hbm.at[pl.ds(base, bpw)], idx_v)
        bufs, sems = (r0, r1), (s0, s1)

        def fire(c):
            return pltpu.async_copy(
                table_hbm.at[idx_v.at[pl.ds(c * chunk, chunk)]],
                bufs[c % 2], sems[c % 2])

        descs = {0: fire(0)}
        if n_chunks > 1:
            descs[1] = fire(1)
        for c in range(n_chunks):
            descs[c].wait()
            pltpu.sync_copy(bufs[c % 2], out_hbm.at[pl.ds(base + c * chunk, chunk)])
            if c + 2 < n_chunks:
                descs[c + 2] = fire(c + 2)

    return gather


def _sc_gather_rows(table, idx, n_rows):
    return _make_sc_gather(n_rows)(table, idx)


def _make_sc_dispatch():
    bpw = A // NW
    n_chunks = 4
    chunk = bpw // n_chunks
    mesh = plsc.VectorSubcoreMesh(core_axis_name="c", subcore_axis_name="s")

    @functools.partial(
        pl.kernel,
        mesh=mesh,
        out_type=jax.ShapeDtypeStruct((PAD, D_MODEL), jnp.float32),
        scratch_types=[
            pltpu.VMEM((chunk,), jnp.int32),
            pltpu.VMEM((chunk,), jnp.int32),
            pltpu.VMEM((chunk, D_MODEL), jnp.float32),
            pltpu.VMEM((chunk, D_MODEL), jnp.float32),
            pltpu.SemaphoreType.DMA,
            pltpu.SemaphoreType.DMA,
        ],
    )
    def dispatch(x_hbm, slots_hbm, out_hbm, i0, i1, r0, r1, s0, s1):
        wid = jax.lax.axis_index("s") * NC + jax.lax.axis_index("c")
        a_base = wid * bpw
        tok_base = jax.lax.rem(a_base, T)
        ibufs, rbufs, sems = (i0, i1), (r0, r1), (s0, s1)

        def load_and_fire(c):
            ib, rb, sem = ibufs[c % 2], rbufs[c % 2], sems[c % 2]
            pltpu.sync_copy(slots_hbm.at[pl.ds(a_base + c * chunk, chunk)], ib)
            pltpu.sync_copy(x_hbm.at[pl.ds(tok_base + c * chunk, chunk)], rb)
            return pltpu.async_copy(rb, out_hbm.at[ib], sem)

        descs = {0: load_and_fire(0), 1: load_and_fire(1)}
        for c in range(n_chunks):
            descs[c].wait()
            if c + 2 < n_chunks:
                descs[c + 2] = load_and_fire(c + 2)

    return dispatch


def _ffn_body(eot_ref, x_ref, w1_ref, b1_ref, w2_ref, b2_ref, out_ref):
    f = pl.program_id(0)
    mt = pl.program_id(1)

    @pl.when(eot_ref[mt] < E)
    def _():
        x = x_ref[...]
        h = jnp.dot(x, w1_ref[0], preferred_element_type=jnp.float32) + b1_ref[0]
        h = _gelu_exact(h)
        contrib = jnp.dot(h, w2_ref[0], preferred_element_type=jnp.float32)
        contrib = contrib + jnp.where(f == 0, 1.0, 0.0) * b2_ref[0]
        sl = pl.ds(mt * TM, TM)
        prev = jnp.where(f == 0, 0.0, out_ref[sl, :])
        out_ref[sl, :] = prev + contrib


def _run_ffn(x_pad, W1, b13, W2, b23, eot):
    def _e(eot_ref, mt):
        return jnp.minimum(eot_ref[mt], E - 1)

    grid_spec = pltpu.PrefetchScalarGridSpec(
        num_scalar_prefetch=1,
        grid=(NF, NT),
        in_specs=[
            pl.BlockSpec((TM, D_MODEL), lambda f, mt, eot: (mt, 0)),
            pl.BlockSpec((1, D_MODEL, TF), lambda f, mt, eot: (_e(eot, mt), 0, f)),
            pl.BlockSpec((1, 1, TF), lambda f, mt, eot: (_e(eot, mt), 0, f)),
            pl.BlockSpec((1, TF, D_MODEL), lambda f, mt, eot: (_e(eot, mt), f, 0)),
            pl.BlockSpec((1, 1, D_MODEL), lambda f, mt, eot: (_e(eot, mt), 0, 0)),
        ],
        out_specs=pl.BlockSpec((PAD, D_MODEL), lambda f, mt, eot: (0, 0)),
    )
    return pl.pallas_call(
        _ffn_body,
        grid_spec=grid_spec,
        out_shape=jax.ShapeDtypeStruct((PAD, D_MODEL), jnp.float32),
        compiler_params=pltpu.CompilerParams(
            dimension_semantics=("arbitrary", "arbitrary"),
        ),
    )(eot, x_pad, W1, b13, W2, b23)


def _combine_body(y0_ref, y1_ref, g0_ref, g1_ref, out_ref):
    out_ref[...] = g0_ref[...] * y0_ref[...] + g1_ref[...] * y1_ref[...]


def _run_combine(yg, g0, g1):
    nt = T // TM
    return pl.pallas_call(
        _combine_body,
        grid=(nt,),
        out_shape=jax.ShapeDtypeStruct((T, D_MODEL), jnp.float32),
        in_specs=[
            pl.BlockSpec((TM, D_MODEL), lambda mt: (mt, 0)),
            pl.BlockSpec((TM, D_MODEL), lambda mt: (mt + nt, 0)),
            pl.BlockSpec((TM, 1), lambda mt: (mt, 0)),
            pl.BlockSpec((TM, 1), lambda mt: (mt, 0)),
        ],
        out_specs=pl.BlockSpec((TM, D_MODEL), lambda mt: (mt, 0)),
    )(yg, yg, g0, g1)


def kernel(x, Wr, br, W1, b1, W2, b2):
    bsz, seq_len, d_model = x.shape
    flat_x = x.reshape(-1, d_model)

    slots32, g0, g1, eot128 = _run_router(flat_x, Wr, br.reshape(1, E))

    x_pad = _make_sc_dispatch()(flat_x, slots32.reshape(A))
    y_pad = _run_ffn(x_pad, W1, b1.reshape(E, 1, D_FF),
                     W2, b2.reshape(E, 1, D_MODEL),
                     eot128.reshape(128))

    yg = _sc_gather_rows(y_pad, slots32.reshape(A), A)
    out = _run_combine(yg, g0, g1)
    return out.reshape(bsz, seq_len, d_model)
